# Initial kernel scaffold; baseline (speedup 1.0000x reference)
import functools

import jax
import jax.numpy as jnp
from jax import lax
from jax.experimental import pallas as pl
from jax.experimental.pallas import tpu as pltpu
from jax.experimental.pallas import tpu_sc as plsc

N = 10000
E = 320000
D = 128

NC = 2
NS = 16
NW = NC * NS
EPT = E // NW
CHUNK = 125
NCHUNK = EPT // CHUNK
RPS = N // NS
DEGW = 16

_MESH = plsc.VectorSubcoreMesh(core_axis_name="c", subcore_axis_name="s")


@functools.partial(
    pl.kernel,
    out_type=jax.ShapeDtypeStruct((NC, N, DEGW), jnp.float32),
    mesh=_MESH,
    scratch_types=[
        pltpu.VMEM((NCHUNK, CHUNK), jnp.int32),
        pltpu.VMEM((CHUNK, DEGW), jnp.float32),
        pltpu.VMEM((CHUNK, DEGW), jnp.float32),
        pltpu.VMEM_SHARED((N, DEGW), jnp.float32),
    ],
)
def _sc_degree(dst_hbm, out_hbm, didx_v, ones_v, zeros_v, acc):
    c = lax.axis_index("c")
    s = lax.axis_index("s")
    wid = c * NS + s

    def fill(i, _):
        ones_v[i, :] = jnp.ones((16,), jnp.float32)
        zeros_v[i, :] = jnp.zeros((16,), jnp.float32)
        return 0

    lax.fori_loop(0, CHUNK, fill, 0)

    for t in range(RPS // CHUNK):
        pltpu.sync_copy(zeros_v, acc.at[pl.ds(s * RPS + t * CHUNK, CHUNK)])
    plsc.subcore_barrier()

    pltpu.sync_copy(dst_hbm.at[wid], didx_v)

    def body(j, _):
        pltpu.sync_copy(ones_v, acc.at[didx_v.at[j]], add=True)
        return 0

    lax.fori_loop(0, NCHUNK, body, 0)
    plsc.subcore_barrier()

    pltpu.sync_copy(acc.at[pl.ds(s * RPS, RPS)], out_hbm.at[c, pl.ds(s * RPS, RPS)])


@functools.partial(
    pl.kernel,
    out_type=jax.ShapeDtypeStruct((NC, N, D), jnp.float32),
    mesh=_MESH,
    scratch_types=[
        pltpu.VMEM((NCHUNK, CHUNK), jnp.int32),
        pltpu.VMEM((NCHUNK, CHUNK), jnp.int32),
        pltpu.VMEM((CHUNK, D), jnp.float32),
        pltpu.VMEM((CHUNK, D), jnp.float32),
        pltpu.VMEM_SHARED((N, D), jnp.float32),
        pltpu.SemaphoreType.DMA,
        pltpu.SemaphoreType.DMA,
    ],
)
def _sc_aggregate(g_hbm, src_hbm, dst_hbm, out_hbm,
                  sidx_v, didx_v, rows0, rows1, acc, sem0, sem1):
    c = lax.axis_index("c")
    s = lax.axis_index("s")
    wid = c * NS + s

    def fill(i, _):
        for k in range(D // 16):
            rows0[i, pl.ds(k * 16, 16)] = jnp.zeros((16,), jnp.float32)
        return 0

    lax.fori_loop(0, CHUNK, fill, 0)
    for t in range(RPS // CHUNK):
        pltpu.sync_copy(rows0, acc.at[pl.ds(s * RPS + t * CHUNK, CHUNK)])
    plsc.subcore_barrier()

    pltpu.sync_copy(src_hbm.at[wid], sidx_v)
    pltpu.sync_copy(dst_hbm.at[wid], didx_v)

    bufs = ((rows0, sem0), (rows1, sem1))
    pltpu.async_copy(g_hbm.at[sidx_v.at[0]], rows0, sem0)
    pltpu.async_copy(g_hbm.at[sidx_v.at[1]], rows1, sem1)

    def body(i, _):
        t = 2 * i
        for b, (rb, sb) in enumerate(bufs):
            j = t + b
            pltpu.make_async_copy(g_hbm.at[sidx_v.at[j]], rb, sb).wait()
            pltpu.sync_copy(rb, acc.at[didx_v.at[j]], add=True)
            pltpu.async_copy(g_hbm.at[sidx_v.at[j + 2]], rb, sb)
        return 0

    lax.fori_loop(0, NCHUNK // 2 - 1, body, 0)

    for b, (rb, sb) in enumerate(bufs):
        j = NCHUNK - 2 + b
        pltpu.make_async_copy(g_hbm.at[sidx_v.at[j]], rb, sb).wait()
        pltpu.sync_copy(rb, acc.at[didx_v.at[j]], add=True)

    plsc.subcore_barrier()
    pltpu.sync_copy(acc.at[pl.ds(s * RPS, RPS)], out_hbm.at[c, pl.ds(s * RPS, RPS)])


_BLK = 1000


def _tc_scale_matmul(x, W, degp):
    def body(x_ref, w_ref, d_ref, g_ref):
        deg = d_ref[0, :, 0:1] + d_ref[1, :, 0:1] + 1.0
        dinv = lax.rsqrt(deg)
        h = jnp.dot(x_ref[...], w_ref[...], preferred_element_type=jnp.float32)
        g_ref[...] = h * dinv

    return pl.pallas_call(
        body,
        grid=(N // _BLK,),
        in_specs=[
            pl.BlockSpec((_BLK, D), lambda i: (i, 0)),
            pl.BlockSpec((D, D), lambda i: (0, 0)),
            pl.BlockSpec((NC, _BLK, DEGW), lambda i: (0, i, 0)),
        ],
        out_specs=pl.BlockSpec((_BLK, D), lambda i: (i, 0)),
        out_shape=jax.ShapeDtypeStruct((N, D), jnp.float32),
    )(x, W, degp)


def _tc_finish(aggp, g, degp, b2):
    def body(a_ref, g_ref, d_ref, b_ref, o_ref):
        deg = d_ref[0, :, 0:1] + d_ref[1, :, 0:1] + 1.0
        dinv = lax.rsqrt(deg)
        logits = (a_ref[0] + a_ref[1] + g_ref[...]) * dinv + b_ref[...]
        o_ref[...] = 1.0 / (1.0 + jnp.exp(-logits))

    return pl.pallas_call(
        body,
        grid=(N // _BLK,),
        in_specs=[
            pl.BlockSpec((NC, _BLK, D), lambda i: (0, i, 0)),
            pl.BlockSpec((_BLK, D), lambda i: (i, 0)),
            pl.BlockSpec((NC, _BLK, DEGW), lambda i: (0, i, 0)),
            pl.BlockSpec((1, D), lambda i: (0, 0)),
        ],
        out_specs=pl.BlockSpec((_BLK, D), lambda i: (i, 0)),
        out_shape=jax.ShapeDtypeStruct((N, D), jnp.float32),
    )(aggp, g, degp, b2)


def kernel(x, edge_index, W, b):
    src_r = edge_index[0].reshape(NW, NCHUNK, CHUNK)
    dst_r = edge_index[1].reshape(NW, NCHUNK, CHUNK)
    degp = _sc_degree(dst_r)
    g = _tc_scale_matmul(x, W, degp)
    aggp = _sc_aggregate(g, src_r, dst_r)
    return _tc_finish(aggp, g, degp, b.reshape(1, D))


# speedup vs baseline: 39.7163x; 39.7163x over previous
import functools

import jax
import jax.numpy as jnp
from jax import lax
from jax.experimental import pallas as pl
from jax.experimental.pallas import tpu as pltpu
from jax.experimental.pallas import tpu_sc as plsc

N = 10000
E = 320000
D = 128

NC = 2
NS = 16
NW = NC * NS
EPT = E // NW
CHUNK = 100
NCHUNK = EPT // CHUNK
NSLAB = 2
SROWS = NCHUNK // NSLAB
DEGW = 16
WB_SUBS = 10
WB_ROWS = N // WB_SUBS

_MESH = plsc.VectorSubcoreMesh(core_axis_name="c", subcore_axis_name="s")


@functools.partial(
    pl.kernel,
    out_type=jax.ShapeDtypeStruct((NC, N, DEGW), jnp.float32),
    mesh=_MESH,
    scratch_types=[
        pltpu.VMEM((SROWS, CHUNK), jnp.int32),
        pltpu.VMEM((CHUNK, DEGW), jnp.float32),
        pltpu.VMEM((CHUNK, DEGW), jnp.float32),
        pltpu.VMEM_SHARED((N, DEGW), jnp.float32),
    ],
)
def _sc_degree(dst_hbm, out_hbm, didx_v, ones_v, zeros_v, acc):
    c = lax.axis_index("c")
    s = lax.axis_index("s")
    wid = c * NS + s

    def fill(i, _):
        ones_v[i, :] = jnp.ones((16,), jnp.float32)
        return 0

    lax.fori_loop(0, CHUNK, fill, 0)

    def zfill(i, _):
        zeros_v[i, :] = jnp.zeros((16,), jnp.float32)
        return 0

    lax.fori_loop(0, CHUNK, zfill, 0)

    @pl.when(s < WB_SUBS)
    def _():
        for t in range(WB_ROWS // CHUNK):
            pltpu.sync_copy(zeros_v, acc.at[pl.ds(s * WB_ROWS + t * CHUNK, CHUNK)])

    plsc.subcore_barrier()

    def body(j, _):
        pltpu.sync_copy(ones_v, acc.at[didx_v.at[j]], add=True)
        return 0

    for half in range(NSLAB):
        pltpu.sync_copy(dst_hbm.at[wid * NSLAB + half], didx_v)
        lax.fori_loop(0, SROWS, body, 0)
    plsc.subcore_barrier()

    @pl.when(s < WB_SUBS)
    def _():
        pltpu.sync_copy(acc.at[pl.ds(s * WB_ROWS, WB_ROWS)],
                        out_hbm.at[c, pl.ds(s * WB_ROWS, WB_ROWS)])


@functools.partial(
    pl.kernel,
    out_type=jax.ShapeDtypeStruct((NC, N, D), jnp.float32),
    mesh=_MESH,
    scratch_types=[
        pltpu.VMEM((SROWS, CHUNK), jnp.int32),
        pltpu.VMEM((SROWS, CHUNK), jnp.int32),
        pltpu.VMEM((CHUNK, D), jnp.float32),
        pltpu.VMEM((CHUNK, D), jnp.float32),
        pltpu.VMEM_SHARED((N, D), jnp.float32),
        pltpu.SemaphoreType.DMA,
        pltpu.SemaphoreType.DMA,
    ],
)
def _sc_aggregate(g_hbm, src_hbm, dst_hbm, out_hbm,
                  sidx_v, didx_v, rows0, rows1, acc, sem0, sem1):
    c = lax.axis_index("c")
    s = lax.axis_index("s")
    wid = c * NS + s

    def zfill(i, _):
        for k in range(D // 16):
            rows0[i, pl.ds(k * 16, 16)] = jnp.zeros((16,), jnp.float32)
        return 0

    lax.fori_loop(0, CHUNK, zfill, 0)

    @pl.when(s < WB_SUBS)
    def _():
        for t in range(WB_ROWS // CHUNK):
            pltpu.sync_copy(rows0, acc.at[pl.ds(s * WB_ROWS + t * CHUNK, CHUNK)])

    plsc.subcore_barrier()

    bufs = ((rows0, sem0), (rows1, sem1))

    def body(i, _):
        t = 2 * i
        for b, (rb, sb) in enumerate(bufs):
            j = t + b
            pltpu.make_async_copy(g_hbm.at[sidx_v.at[j]], rb, sb).wait()
            pltpu.sync_copy(rb, acc.at[didx_v.at[j]], add=True)
            pltpu.async_copy(g_hbm.at[sidx_v.at[j + 2]], rb, sb)
        return 0

    for half in range(NSLAB):
        slab = wid * NSLAB + half
        pltpu.sync_copy(src_hbm.at[slab], sidx_v)
        pltpu.sync_copy(dst_hbm.at[slab], didx_v)
        pltpu.async_copy(g_hbm.at[sidx_v.at[0]], rows0, sem0)
        pltpu.async_copy(g_hbm.at[sidx_v.at[1]], rows1, sem1)
        lax.fori_loop(0, SROWS // 2 - 1, body, 0)
        for b, (rb, sb) in enumerate(bufs):
            j = SROWS - 2 + b
            pltpu.make_async_copy(g_hbm.at[sidx_v.at[j]], rb, sb).wait()
            pltpu.sync_copy(rb, acc.at[didx_v.at[j]], add=True)

    plsc.subcore_barrier()

    @pl.when(s < WB_SUBS)
    def _():
        pltpu.sync_copy(acc.at[pl.ds(s * WB_ROWS, WB_ROWS)],
                        out_hbm.at[c, pl.ds(s * WB_ROWS, WB_ROWS)])


_BLK = 1000


def _tc_scale_matmul(x, W, degp):
    def body(x_ref, w_ref, d_ref, g_ref):
        deg = d_ref[0, :, 0:1] + d_ref[1, :, 0:1] + 1.0
        dinv = lax.rsqrt(deg)
        h = jnp.dot(x_ref[...], w_ref[...], preferred_element_type=jnp.float32)
        g_ref[...] = h * dinv

    return pl.pallas_call(
        body,
        grid=(N // _BLK,),
        in_specs=[
            pl.BlockSpec((_BLK, D), lambda i: (i, 0)),
            pl.BlockSpec((D, D), lambda i: (0, 0)),
            pl.BlockSpec((NC, _BLK, DEGW), lambda i: (0, i, 0)),
        ],
        out_specs=pl.BlockSpec((_BLK, D), lambda i: (i, 0)),
        out_shape=jax.ShapeDtypeStruct((N, D), jnp.float32),
    )(x, W, degp)


def _tc_finish(aggp, g, degp, b2):
    def body(a_ref, g_ref, d_ref, b_ref, o_ref):
        deg = d_ref[0, :, 0:1] + d_ref[1, :, 0:1] + 1.0
        dinv = lax.rsqrt(deg)
        logits = (a_ref[0] + a_ref[1] + g_ref[...]) * dinv + b_ref[...]
        o_ref[...] = 1.0 / (1.0 + jnp.exp(-logits))

    return pl.pallas_call(
        body,
        grid=(N // _BLK,),
        in_specs=[
            pl.BlockSpec((NC, _BLK, D), lambda i: (0, i, 0)),
            pl.BlockSpec((_BLK, D), lambda i: (i, 0)),
            pl.BlockSpec((NC, _BLK, DEGW), lambda i: (0, i, 0)),
            pl.BlockSpec((1, D), lambda i: (0, 0)),
        ],
        out_specs=pl.BlockSpec((_BLK, D), lambda i: (i, 0)),
        out_shape=jax.ShapeDtypeStruct((N, D), jnp.float32),
    )(aggp, g, degp, b2)


def kernel(x, edge_index, W, b):
    src_r = edge_index[0].reshape(NW * NSLAB, SROWS, CHUNK)
    dst_r = edge_index[1].reshape(NW * NSLAB, SROWS, CHUNK)
    degp = _sc_degree(dst_r)
    g = _tc_scale_matmul(x, W, degp)
    aggp = _sc_aggregate(g, src_r, dst_r)
    return _tc_finish(aggp, g, degp, b.reshape(1, D))


# speedup vs baseline: 40.4382x; 1.0182x over previous
import functools

import jax
import jax.numpy as jnp
from jax import lax
from jax.experimental import pallas as pl
from jax.experimental.pallas import tpu as pltpu
from jax.experimental.pallas import tpu_sc as plsc

N = 10000
E = 320000
D = 128

NC = 2
NS = 16
NW = NC * NS
EPT = E // NW
CHUNK = 100
NCHUNK = EPT // CHUNK
NSLAB = 5
SROWS = NCHUNK // NSLAB
DEGW = 16
WB_SUBS = 10
WB_ROWS = N // WB_SUBS

_MESH = plsc.VectorSubcoreMesh(core_axis_name="c", subcore_axis_name="s")


@functools.partial(
    pl.kernel,
    out_type=jax.ShapeDtypeStruct((NC, N, DEGW), jnp.float32),
    mesh=_MESH,
    scratch_types=[
        pltpu.VMEM((SROWS, CHUNK), jnp.int32),
        pltpu.VMEM((CHUNK, DEGW), jnp.float32),
        pltpu.VMEM((CHUNK, DEGW), jnp.float32),
        pltpu.VMEM_SHARED((N, DEGW), jnp.float32),
    ],
)
def _sc_degree(dst_hbm, out_hbm, didx_v, ones_v, zeros_v, acc):
    c = lax.axis_index("c")
    s = lax.axis_index("s")
    wid = c * NS + s

    def fill(i, _):
        ones_v[i, :] = jnp.ones((16,), jnp.float32)
        return 0

    lax.fori_loop(0, CHUNK, fill, 0)

    def zfill(i, _):
        zeros_v[i, :] = jnp.zeros((16,), jnp.float32)
        return 0

    lax.fori_loop(0, CHUNK, zfill, 0)

    @pl.when(s < WB_SUBS)
    def _():
        for t in range(WB_ROWS // CHUNK):
            pltpu.sync_copy(zeros_v, acc.at[pl.ds(s * WB_ROWS + t * CHUNK, CHUNK)])

    plsc.subcore_barrier()

    def body(j, _):
        pltpu.sync_copy(ones_v, acc.at[didx_v.at[j]], add=True)
        return 0

    for half in range(NSLAB):
        pltpu.sync_copy(dst_hbm.at[wid * NSLAB + half], didx_v)
        lax.fori_loop(0, SROWS, body, 0)
    plsc.subcore_barrier()

    @pl.when(s < WB_SUBS)
    def _():
        pltpu.sync_copy(acc.at[pl.ds(s * WB_ROWS, WB_ROWS)],
                        out_hbm.at[c, pl.ds(s * WB_ROWS, WB_ROWS)])


@functools.partial(
    pl.kernel,
    out_type=jax.ShapeDtypeStruct((NC, N, D), jnp.float32),
    mesh=_MESH,
    scratch_types=[
        pltpu.VMEM((SROWS, CHUNK), jnp.int32),
        pltpu.VMEM((SROWS, CHUNK), jnp.int32),
        pltpu.VMEM((CHUNK, D), jnp.float32),
        pltpu.VMEM((CHUNK, D), jnp.float32),
        pltpu.VMEM((CHUNK, D), jnp.float32),
        pltpu.VMEM_SHARED((N, D), jnp.float32),
        pltpu.SemaphoreType.DMA,
        pltpu.SemaphoreType.DMA,
        pltpu.SemaphoreType.DMA,
    ],
)
def _sc_aggregate(g_hbm, src_hbm, dst_hbm, out_hbm,
                  sidx_v, didx_v, rows0, rows1, rows2, acc, sem0, sem1, sem2):
    c = lax.axis_index("c")
    s = lax.axis_index("s")
    wid = c * NS + s

    def zfill(i, _):
        for k in range(D // 16):
            rows0[i, pl.ds(k * 16, 16)] = jnp.zeros((16,), jnp.float32)
        return 0

    lax.fori_loop(0, CHUNK, zfill, 0)

    @pl.when(s < WB_SUBS)
    def _():
        for t in range(WB_ROWS // CHUNK):
            pltpu.sync_copy(rows0, acc.at[pl.ds(s * WB_ROWS + t * CHUNK, CHUNK)])

    plsc.subcore_barrier()

    bufs = ((rows0, sem0), (rows1, sem1), (rows2, sem2))

    def body(i, _):
        t = 3 * i
        for b, (rb, sb) in enumerate(bufs):
            j = t + b
            pltpu.make_async_copy(g_hbm.at[sidx_v.at[j]], rb, sb).wait()
            pltpu.sync_copy(rb, acc.at[didx_v.at[j]], add=True)
            pltpu.async_copy(g_hbm.at[sidx_v.at[j + 3]], rb, sb)
        return 0

    for half in range(NSLAB):
        slab = wid * NSLAB + half
        pltpu.sync_copy(src_hbm.at[slab], sidx_v)
        pltpu.sync_copy(dst_hbm.at[slab], didx_v)
        pltpu.async_copy(g_hbm.at[sidx_v.at[0]], rows0, sem0)
        pltpu.async_copy(g_hbm.at[sidx_v.at[1]], rows1, sem1)
        pltpu.async_copy(g_hbm.at[sidx_v.at[2]], rows2, sem2)
        lax.fori_loop(0, (SROWS - 5) // 3, body, 0)
        for b, (rb, sb) in enumerate(bufs):
            j = SROWS - 5 + b
            pltpu.make_async_copy(g_hbm.at[sidx_v.at[j]], rb, sb).wait()
            pltpu.sync_copy(rb, acc.at[didx_v.at[j]], add=True)
            if j + 3 < SROWS:
                pltpu.async_copy(g_hbm.at[sidx_v.at[j + 3]], rb, sb)
        for b, (rb, sb) in enumerate(bufs):
            j = SROWS - 2 + b
            if j < SROWS:
                pltpu.make_async_copy(g_hbm.at[sidx_v.at[j]], rb, sb).wait()
                pltpu.sync_copy(rb, acc.at[didx_v.at[j]], add=True)

    plsc.subcore_barrier()

    @pl.when(s < WB_SUBS)
    def _():
        pltpu.sync_copy(acc.at[pl.ds(s * WB_ROWS, WB_ROWS)],
                        out_hbm.at[c, pl.ds(s * WB_ROWS, WB_ROWS)])


_BLK = 1000


def _tc_scale_matmul(x, W, degp):
    def body(x_ref, w_ref, d_ref, g_ref):
        deg = d_ref[0, :, 0:1] + d_ref[1, :, 0:1] + 1.0
        dinv = lax.rsqrt(deg)
        h = jnp.dot(x_ref[...], w_ref[...], preferred_element_type=jnp.float32)
        g_ref[...] = h * dinv

    return pl.pallas_call(
        body,
        grid=(N // _BLK,),
        in_specs=[
            pl.BlockSpec((_BLK, D), lambda i: (i, 0)),
            pl.BlockSpec((D, D), lambda i: (0, 0)),
            pl.BlockSpec((NC, _BLK, DEGW), lambda i: (0, i, 0)),
        ],
        out_specs=pl.BlockSpec((_BLK, D), lambda i: (i, 0)),
        out_shape=jax.ShapeDtypeStruct((N, D), jnp.float32),
    )(x, W, degp)


def _tc_finish(aggp, g, degp, b2):
    def body(a_ref, g_ref, d_ref, b_ref, o_ref):
        deg = d_ref[0, :, 0:1] + d_ref[1, :, 0:1] + 1.0
        dinv = lax.rsqrt(deg)
        logits = (a_ref[0] + a_ref[1] + g_ref[...]) * dinv + b_ref[...]
        o_ref[...] = 1.0 / (1.0 + jnp.exp(-logits))

    return pl.pallas_call(
        body,
        grid=(N // _BLK,),
        in_specs=[
            pl.BlockSpec((NC, _BLK, D), lambda i: (0, i, 0)),
            pl.BlockSpec((_BLK, D), lambda i: (i, 0)),
            pl.BlockSpec((NC, _BLK, DEGW), lambda i: (0, i, 0)),
            pl.BlockSpec((1, D), lambda i: (0, 0)),
        ],
        out_specs=pl.BlockSpec((_BLK, D), lambda i: (i, 0)),
        out_shape=jax.ShapeDtypeStruct((N, D), jnp.float32),
    )(aggp, g, degp, b2)


def kernel(x, edge_index, W, b):
    src_r = edge_index[0].reshape(NW * NSLAB, SROWS, CHUNK)
    dst_r = edge_index[1].reshape(NW * NSLAB, SROWS, CHUNK)
    degp = _sc_degree(dst_r)
    g = _tc_scale_matmul(x, W, degp)
    aggp = _sc_aggregate(g, src_r, dst_r)
    return _tc_finish(aggp, g, degp, b.reshape(1, D))
